# baseline (speedup 1.0000x reference)
import functools

import jax
import jax.numpy as jnp
from jax import lax
from jax.experimental import pallas as pl
from jax.experimental.pallas import tpu as pltpu
from jax.experimental.pallas import tpu_sc as plsc

VOCAB = 100000
DIM = 64
FIELDS = 100
BATCH = 4096

NW = 32
ITEMS_PER_W = BATCH // NW
G = 4
GROUPS_PER_W = ITEMS_PER_W // G
NBUF = 3
LAG = 2


def _make_kernel():
    mesh = plsc.VectorSubcoreMesh(core_axis_name="c", subcore_axis_name="s")

    @functools.partial(
        pl.kernel,
        mesh=mesh,
        out_type=jax.ShapeDtypeStruct((BATCH, FIELDS, DIM), jnp.float32),
        compiler_params=pltpu.CompilerParams(use_tc_tiling_on_sc=False),
        scratch_types=[
            pltpu.VMEM((ITEMS_PER_W, FIELDS), jnp.int32),
            pltpu.VMEM((FIELDS, DIM), jnp.float32),
            pltpu.VMEM((NBUF, G, FIELDS, DIM), jnp.float32),
            pltpu.SemaphoreType.DMA((NBUF,)),
            pltpu.SemaphoreType.DMA((NBUF,)),
        ],
    )
    def k(table_hbm, idx_hbm, pos_hbm, out_hbm, idx_v, pos_v, rows_v, gsem, ssem):
        wid = lax.axis_index("s") * 2 + lax.axis_index("c")
        item_base = wid * ITEMS_PER_W
        pltpu.sync_copy(idx_hbm.at[pl.ds(item_base, ITEMS_PER_W)], idx_v)
        pltpu.sync_copy(pos_hbm, pos_v)

        def gather_copy(g, i, b):
            return pltpu.make_async_copy(
                table_hbm.at[idx_v.at[g * G + i]],
                rows_v.at[b, i],
                gsem.at[b],
            )

        def store_copy(g, b):
            return pltpu.make_async_copy(
                rows_v.at[b],
                out_hbm.at[pl.ds(item_base + g * G, G)],
                ssem.at[b],
            )

        def step(t, carry):
            @pl.when(t < GROUPS_PER_W)
            def _issue():
                b = lax.rem(t, NBUF)

                @pl.when(t >= NBUF)
                def _drain_store():
                    store_copy(t - NBUF, b).wait()

                for i in range(G):
                    gather_copy(t, i, b).start()

            @pl.when(t >= LAG)
            def _process():
                gp = t - LAG
                b = lax.rem(gp, NBUF)
                for i in range(G):
                    gather_copy(gp, i, b).wait()

                @plsc.parallel_loop(0, FIELDS, unroll=4)
                def _add(j):
                    for i in range(G):
                        for q in range(DIM // 16):
                            plsc.addupdate(
                                rows_v.at[b, i, j, pl.ds(q * 16, 16)],
                                pos_v[j, pl.ds(q * 16, 16)],
                            )

                store_copy(gp, b).start()

            return carry

        lax.fori_loop(0, GROUPS_PER_W + LAG, step, 0)

        for g in range(GROUPS_PER_W - NBUF, GROUPS_PER_W):
            store_copy(g, g % NBUF).wait()

    return k


_gather_kernel = _make_kernel()


def kernel(x, cat_embed_weight, pos_encoder):
    idx = x.reshape(BATCH, FIELDS).astype(jnp.int32)
    return _gather_kernel(cat_embed_weight, idx, pos_encoder)
